# baseline (speedup 1.0000x reference)
import functools

import jax
import jax.numpy as jnp
from jax import lax
from jax.experimental import pallas as pl
from jax.experimental.pallas import tpu as pltpu
from jax.experimental.pallas import tpu_sc as plsc

VOCAB = 1000000
D = 64
T = 16384
S = 50

_info = plsc.get_sparse_core_info()
NC, NS = _info.num_cores, _info.num_subcores
NW = NC * NS

ROWS_PER_W = T // NW
CROWS = 16
N_CHUNKS = ROWS_PER_W // CROWS
NBUF = 2


def _make_kernel():
    mesh = plsc.VectorSubcoreMesh(core_axis_name="c", subcore_axis_name="s")

    @functools.partial(
        pl.kernel,
        mesh=mesh,
        out_type=jax.ShapeDtypeStruct((T, S, D), jnp.float32),
        scratch_types=[
            [pltpu.VMEM((CROWS, S), jnp.int32) for _ in range(NBUF)],
            [pltpu.VMEM((CROWS, S, D), jnp.float32) for _ in range(NBUF)],
            pltpu.SemaphoreType.DMA,
            [pltpu.SemaphoreType.DMA for _ in range(NBUF)],
            [pltpu.SemaphoreType.DMA for _ in range(NBUF)],
        ],
        compiler_params=pltpu.CompilerParams(use_tc_tiling_on_sc=False),
    )
    def emb(tok_hbm, w_hbm, out_hbm, idx_v, rows_v, sem_g, sem_i, sem_o):
        wid = lax.axis_index("s") * NC + lax.axis_index("c")
        base = wid * ROWS_PER_W

        def idx_copy(c, b):
            return pltpu.make_async_copy(
                tok_hbm.at[pl.ds(base + c * CROWS, CROWS)], idx_v[b], sem_i[b]
            )

        def out_copy(c, b):
            return pltpu.make_async_copy(
                rows_v[b], out_hbm.at[pl.ds(base + c * CROWS, CROWS)], sem_o[b]
            )

        for b in range(NBUF):
            idx_copy(b, b).start()

        def outer(g, carry):
            for b in range(NBUF):
                i = g * NBUF + b

                @pl.when(g > 0)
                def _wait_rows_free():
                    out_copy(i, b).wait()

                idx_copy(i, b).wait()
                for r in range(CROWS):
                    pltpu.async_copy(
                        w_hbm.at[idx_v[b].at[r]], rows_v[b].at[r], sem_g
                    )
                for r in range(CROWS):
                    pltpu.make_async_copy(
                        w_hbm.at[idx_v[b].at[r]], rows_v[b].at[r], sem_g
                    ).wait()
                out_copy(i, b).start()

                @pl.when(i + NBUF < N_CHUNKS)
                def _prefetch_idx():
                    idx_copy(i + NBUF, b).start()

            return carry

        lax.fori_loop(0, N_CHUNKS // NBUF, outer, 0)
        for b in range(NBUF):
            out_copy(N_CHUNKS - NBUF + b, b).wait()

    return emb


_emb = _make_kernel()


@jax.jit
def kernel(token_ids, W):
    return _emb(token_ids, W)


# speedup vs baseline: 1.3464x; 1.3464x over previous
import functools

import jax
import jax.numpy as jnp
from jax import lax
from jax.experimental import pallas as pl
from jax.experimental.pallas import tpu as pltpu
from jax.experimental.pallas import tpu_sc as plsc

VOCAB = 1000000
D = 64
T = 16384
S = 50
SP = 56
DP = 128

_info = plsc.get_sparse_core_info()
NC, NS = _info.num_cores, _info.num_subcores
NW = NC * NS

ROWS_PER_W = T // NW
CROWS = 16
N_CHUNKS = ROWS_PER_W // CROWS
NBUF = 2


def _make_kernel():
    mesh = plsc.VectorSubcoreMesh(core_axis_name="c", subcore_axis_name="s")

    @functools.partial(
        pl.kernel,
        mesh=mesh,
        out_type=jax.ShapeDtypeStruct((T, SP, DP), jnp.float32),
        scratch_types=[
            [pltpu.VMEM((CROWS, S), jnp.int32) for _ in range(NBUF)],
            [pltpu.VMEM((CROWS, S, D), jnp.float32) for _ in range(NBUF)],
            pltpu.SemaphoreType.DMA,
            [pltpu.SemaphoreType.DMA for _ in range(NBUF)],
            [pltpu.SemaphoreType.DMA for _ in range(NBUF)],
        ],
        compiler_params=pltpu.CompilerParams(use_tc_tiling_on_sc=False),
    )
    def emb(tok_hbm, w_hbm, out_hbm, idx_v, rows_v, sem_g, sem_i, sem_o):
        wid = lax.axis_index("s") * NC + lax.axis_index("c")
        base = wid * ROWS_PER_W

        def idx_copy(c, b):
            return pltpu.make_async_copy(
                tok_hbm.at[pl.ds(base + c * CROWS, CROWS)], idx_v[b], sem_i[b]
            )

        def out_copy(c, b):
            return pltpu.make_async_copy(
                rows_v[b],
                out_hbm.at[
                    pl.ds(base + c * CROWS, CROWS), pl.ds(0, S), pl.ds(0, D)
                ],
                sem_o[b],
            )

        for b in range(NBUF):
            idx_copy(b, b).start()

        def outer(g, carry):
            for b in range(NBUF):
                i = g * NBUF + b

                @pl.when(g > 0)
                def _wait_rows_free():
                    out_copy(i, b).wait()

                idx_copy(i, b).wait()
                for r in range(CROWS):
                    pltpu.async_copy(
                        w_hbm.at[idx_v[b].at[r]], rows_v[b].at[r], sem_g
                    )
                for r in range(CROWS):
                    pltpu.make_async_copy(
                        w_hbm.at[idx_v[b].at[r]], rows_v[b].at[r], sem_g
                    ).wait()
                out_copy(i, b).start()

                @pl.when(i + NBUF < N_CHUNKS)
                def _prefetch_idx():
                    idx_copy(i + NBUF, b).start()

            return carry

        lax.fori_loop(0, N_CHUNKS // NBUF, outer, 0)
        for b in range(NBUF):
            out_copy(N_CHUNKS - NBUF + b, b).wait()

    return emb


_emb = _make_kernel()


@jax.jit
def kernel(token_ids, W):
    out = _emb(token_ids, W)
    return out[:, :S, :D]


# speedup vs baseline: 1.3488x; 1.0018x over previous
import functools

import jax
import jax.numpy as jnp
from jax import lax
from jax.experimental import pallas as pl
from jax.experimental.pallas import tpu as pltpu
from jax.experimental.pallas import tpu_sc as plsc

VOCAB = 1000000
D = 64
T = 16384
S = 50
SP = 56
DP = 128

_info = plsc.get_sparse_core_info()
NC, NS = _info.num_cores, _info.num_subcores
NW = NC * NS

ROWS_PER_W = T // NW
CROWS = 8
N_CHUNKS = ROWS_PER_W // CROWS
NBUF = 4


def _make_kernel():
    mesh = plsc.VectorSubcoreMesh(core_axis_name="c", subcore_axis_name="s")

    @functools.partial(
        pl.kernel,
        mesh=mesh,
        out_type=jax.ShapeDtypeStruct((T, SP, DP), jnp.float32),
        scratch_types=[
            [pltpu.VMEM((CROWS, S), jnp.int32) for _ in range(NBUF)],
            [pltpu.VMEM((CROWS, S, D), jnp.float32) for _ in range(NBUF)],
            [pltpu.SemaphoreType.DMA for _ in range(NBUF)],
            [pltpu.SemaphoreType.DMA for _ in range(NBUF)],
            [pltpu.SemaphoreType.DMA for _ in range(NBUF)],
        ],
        compiler_params=pltpu.CompilerParams(use_tc_tiling_on_sc=False),
    )
    def emb(tok_hbm, w_hbm, out_hbm, idx_v, rows_v, sem_g, sem_i, sem_o):
        wid = lax.axis_index("s") * NC + lax.axis_index("c")
        base = wid * ROWS_PER_W

        def idx_copy(c, b):
            return pltpu.make_async_copy(
                tok_hbm.at[pl.ds(base + c * CROWS, CROWS)], idx_v[b], sem_i[b]
            )

        def out_copy(c, b):
            return pltpu.make_async_copy(
                rows_v[b],
                out_hbm.at[
                    pl.ds(base + c * CROWS, CROWS), pl.ds(0, S), pl.ds(0, D)
                ],
                sem_o[b],
            )

        def fire_gathers(b):
            for r in range(CROWS):
                pltpu.async_copy(
                    w_hbm.at[idx_v[b].at[r]], rows_v[b].at[r], sem_g[b]
                )

        def drain_gathers(b):
            for r in range(CROWS):
                pltpu.make_async_copy(
                    w_hbm.at[idx_v[b].at[r]], rows_v[b].at[r], sem_g[b]
                ).wait()

        for b in range(NBUF):
            idx_copy(b, b).start()

        def outer(g, carry):
            for b in range(NBUF):
                i = g * NBUF + b
                pb = (b - 1) % NBUF

                @pl.when(g > 0)
                def _wait_rows_free():
                    out_copy(i - NBUF, b).wait()

                idx_copy(i, b).wait()
                fire_gathers(b)

                @pl.when(i > 0)
                def _retire_prev():
                    drain_gathers(pb)
                    out_copy(i - 1, pb).start()

                    @pl.when(i + NBUF - 1 < N_CHUNKS)
                    def _prefetch_idx():
                        idx_copy(i + NBUF - 1, pb).start()

            return carry

        lax.fori_loop(0, N_CHUNKS // NBUF, outer, 0)
        last_b = (N_CHUNKS - 1) % NBUF
        drain_gathers(last_b)
        out_copy(N_CHUNKS - 1, last_b).start()
        for b in range(NBUF):
            out_copy(N_CHUNKS - NBUF + b, b).wait()

    return emb


_emb = _make_kernel()


@jax.jit
def kernel(token_ids, W):
    out = _emb(token_ids, W)
    return out[:, :S, :D]
